# baseline (speedup 1.0000x reference)
import jax
import jax.numpy as jnp
from jax.experimental import pallas as pl
from jax.experimental.pallas import tpu as pltpu

N = 10000
F = 128
BLOCK = 1000


def _body(x_ref, w1_ref, b1_ref, w2_ref, b2_ref, o_ref):
    h = jnp.dot(x_ref[...], w1_ref[...], preferred_element_type=jnp.float32)
    h = jnp.maximum(h + b1_ref[...], 0.0)
    out = jnp.dot(h, w2_ref[...], preferred_element_type=jnp.float32)
    out = out + b2_ref[...]
    m = jnp.max(out, axis=-1, keepdims=True)
    e = jnp.exp(out - m)
    s = jnp.sum(e, axis=-1, keepdims=True)
    o_ref[...] = out - (m + jnp.log(s))


def kernel(x, W1, b1, W2, b2, edge_index):
    del edge_index
    w1t = W1.T
    w2t = W2.T
    b1r = b1.reshape(1, F)
    b2r = b2.reshape(1, F)
    grid = (N // BLOCK,)
    return pl.pallas_call(
        _body,
        grid=grid,
        in_specs=[
            pl.BlockSpec((BLOCK, F), lambda i: (i, 0)),
            pl.BlockSpec((F, F), lambda i: (0, 0)),
            pl.BlockSpec((1, F), lambda i: (0, 0)),
            pl.BlockSpec((F, F), lambda i: (0, 0)),
            pl.BlockSpec((1, F), lambda i: (0, 0)),
        ],
        out_specs=pl.BlockSpec((BLOCK, F), lambda i: (i, 0)),
        out_shape=jax.ShapeDtypeStruct((N, F), jnp.float32),
        compiler_params=pltpu.CompilerParams(
            dimension_semantics=("parallel",),
        ),
    )(x, w1t, b1r, w2t, b2r)


# speedup vs baseline: 1.2512x; 1.2512x over previous
import jax
import jax.numpy as jnp
from jax.experimental import pallas as pl
from jax.experimental.pallas import tpu as pltpu

N = 10000
F = 128
BLOCK = 2000


def _body(x_ref, w1_ref, b1_ref, w2_ref, b2_ref, o_ref):
    h = jnp.dot(x_ref[...], w1_ref[...], preferred_element_type=jnp.float32)
    h = jnp.maximum(h + b1_ref[...], 0.0)
    out = jnp.dot(h, w2_ref[...], preferred_element_type=jnp.float32)
    out = out + b2_ref[...]
    m = jnp.max(out, axis=-1, keepdims=True)
    e = jnp.exp(out - m)
    s = jnp.sum(e, axis=-1, keepdims=True)
    o_ref[...] = out - (m + jnp.log(s))


def kernel(x, W1, b1, W2, b2, edge_index):
    del edge_index
    w1t = W1.T
    w2t = W2.T
    b1r = b1.reshape(1, F)
    b2r = b2.reshape(1, F)
    grid = (N // BLOCK,)
    return pl.pallas_call(
        _body,
        grid=grid,
        in_specs=[
            pl.BlockSpec((BLOCK, F), lambda i: (i, 0)),
            pl.BlockSpec((F, F), lambda i: (0, 0)),
            pl.BlockSpec((1, F), lambda i: (0, 0)),
            pl.BlockSpec((F, F), lambda i: (0, 0)),
            pl.BlockSpec((1, F), lambda i: (0, 0)),
        ],
        out_specs=pl.BlockSpec((BLOCK, F), lambda i: (i, 0)),
        out_shape=jax.ShapeDtypeStruct((N, F), jnp.float32),
        compiler_params=pltpu.CompilerParams(
            dimension_semantics=("parallel",),
        ),
    )(x, w1t, b1r, w2t, b2r)


# speedup vs baseline: 1.4441x; 1.1542x over previous
import jax
import jax.numpy as jnp
from jax.experimental import pallas as pl
from jax.experimental.pallas import tpu as pltpu

N = 10000
F = 128
BLOCK = 5000


def _body(x_ref, w1_ref, b1_ref, w2_ref, b2_ref, o_ref):
    h = jnp.dot(x_ref[...], w1_ref[...], preferred_element_type=jnp.float32)
    h = jnp.maximum(h + b1_ref[...], 0.0)
    out = jnp.dot(h, w2_ref[...], preferred_element_type=jnp.float32)
    out = out + b2_ref[...]
    m = jnp.max(out, axis=-1, keepdims=True)
    e = jnp.exp(out - m)
    s = jnp.sum(e, axis=-1, keepdims=True)
    o_ref[...] = out - (m + jnp.log(s))


def kernel(x, W1, b1, W2, b2, edge_index):
    del edge_index
    w1t = W1.T
    w2t = W2.T
    b1r = b1.reshape(1, F)
    b2r = b2.reshape(1, F)
    grid = (N // BLOCK,)
    return pl.pallas_call(
        _body,
        grid=grid,
        in_specs=[
            pl.BlockSpec((BLOCK, F), lambda i: (i, 0)),
            pl.BlockSpec((F, F), lambda i: (0, 0)),
            pl.BlockSpec((1, F), lambda i: (0, 0)),
            pl.BlockSpec((F, F), lambda i: (0, 0)),
            pl.BlockSpec((1, F), lambda i: (0, 0)),
        ],
        out_specs=pl.BlockSpec((BLOCK, F), lambda i: (i, 0)),
        out_shape=jax.ShapeDtypeStruct((N, F), jnp.float32),
        compiler_params=pltpu.CompilerParams(
            dimension_semantics=("parallel",),
        ),
    )(x, w1t, b1r, w2t, b2r)
